# baseline (speedup 1.0000x reference)
import functools

import jax
import jax.numpy as jnp
from jax import lax
from jax.experimental import pallas as pl
from jax.experimental.pallas import tpu as pltpu
from jax.experimental.pallas import tpu_sc as plsc

EMBED_DIM = 64
NUM_CORES = 2
NUM_SUBCORES = 16
NUM_WORKERS = NUM_CORES * NUM_SUBCORES
LANES = 16

BCHUNK = 256


@functools.lru_cache(maxsize=None)
def _make_gather(batch: int, hist: int):
    bpw = batch // NUM_WORKERS
    cpw = bpw // BCHUNK
    num_chunks = hist * cpw
    assert batch % (NUM_WORKERS * BCHUNK) == 0
    mesh = plsc.VectorSubcoreMesh(core_axis_name="c", subcore_axis_name="s")

    @functools.partial(
        pl.kernel,
        mesh=mesh,
        out_type=jax.ShapeDtypeStruct((hist, EMBED_DIM, batch), jnp.float32),
        compiler_params=pltpu.CompilerParams(use_tc_tiling_on_sc=False,
                                             needs_layout_passes=False),
        scratch_types=[
            pltpu.VMEM((BCHUNK,), jnp.int32),
            pltpu.VMEM((BCHUNK,), jnp.int32),
            pltpu.VMEM((BCHUNK, EMBED_DIM), jnp.float32),
            pltpu.VMEM((BCHUNK, EMBED_DIM), jnp.float32),
            pltpu.VMEM((EMBED_DIM, BCHUNK), jnp.float32),
            pltpu.VMEM((EMBED_DIM, BCHUNK), jnp.float32),
            pltpu.SemaphoreType.DMA,
            pltpu.SemaphoreType.DMA,
            pltpu.SemaphoreType.DMA,
            pltpu.SemaphoreType.DMA,
            pltpu.SemaphoreType.DMA,
            pltpu.SemaphoreType.DMA,
        ],
    )
    def gather_kernel(idx_hbm, table_hbm, out_hbm,
                      idx0, idx1, rows0, rows1, tr0, tr1,
                      isem0, isem1, gsem0, gsem1, osem0, osem1):
        wid = lax.axis_index("s") * NUM_CORES + lax.axis_index("c")
        base = wid * bpw

        def chunk_off(i):
            h = lax.div(i, cpw)
            bo = base + lax.rem(i, cpw) * BCHUNK
            return h, bo

        bufs = ((idx0, rows0, tr0, isem0, gsem0, osem0),
                (idx1, rows1, tr1, isem1, gsem1, osem1))

        rowidx = [lax.iota(jnp.int32, LANES) + LANES * k
                  for k in range(BCHUNK // LANES)]

        h0, bo0 = chunk_off(0)
        pltpu.sync_copy(idx_hbm.at[h0, pl.ds(bo0, BCHUNK)], idx0)
        pltpu.async_copy(table_hbm.at[idx0], rows0, gsem0)
        h1, bo1 = chunk_off(1)
        pltpu.async_copy(idx_hbm.at[h1, pl.ds(bo1, BCHUNK)], idx1, isem1)

        def body(i, _):
            cur = lax.rem(i, 2)
            for b, (idx_v, rows_v, tr_v, isem, gsem, osem) in enumerate(bufs):
                @pl.when(cur == b)
                def _():
                    nidx_v, nrows_v, _, nisem, ngsem, _ = bufs[1 - b]
                    h, bo = chunk_off(i)
                    pltpu.make_async_copy(table_hbm.at[idx_v], rows_v,
                                          gsem).wait()
                    @pl.when(i + 1 < num_chunks)
                    def _():
                        hn, bon = chunk_off(i + 1)
                        pltpu.make_async_copy(
                            idx_hbm.at[hn, pl.ds(bon, BCHUNK)],
                            nidx_v, nisem).wait()
                        pltpu.async_copy(table_hbm.at[nidx_v], nrows_v, ngsem)
                    @pl.when(i + 2 < num_chunks)
                    def _():
                        h2, bo2 = chunk_off(i + 2)
                        pltpu.async_copy(idx_hbm.at[h2, pl.ds(bo2, BCHUNK)],
                                         idx_v, isem)
                    @pl.when(i >= 2)
                    def _():
                        hp, bop = chunk_off(i - 2)
                        pltpu.make_async_copy(
                            tr_v, out_hbm.at[hp, :, pl.ds(bop, BCHUNK)],
                            osem).wait()
                    def trans(e, _):
                        cole = jnp.full((LANES,), 0, jnp.int32) + e
                        for k in range(BCHUNK // LANES):
                            v = plsc.load_gather(rows_v, [rowidx[k], cole])
                            tr_v[e, pl.ds(LANES * k, LANES)] = v
                        return 0

                    lax.fori_loop(0, EMBED_DIM, trans, 0)
                    pltpu.async_copy(tr_v, out_hbm.at[h, :, pl.ds(bo, BCHUNK)],
                                     osem)
            return 0

        lax.fori_loop(0, num_chunks, body, 0)

        for last, (_, _, tr_v, _, _, osem) in zip(
                (num_chunks - 2, num_chunks - 1), bufs):
            hl, bol = chunk_off(last)
            pltpu.make_async_copy(tr_v, out_hbm.at[hl, :, pl.ds(bol, BCHUNK)],
                                  osem).wait()

    return gather_kernel


def kernel(x, table):
    batch, hist = x.shape
    idx2 = x.T.astype(jnp.int32)
    out = _make_gather(batch, hist)(idx2, table)
    return jnp.transpose(out, (2, 0, 1))


# speedup vs baseline: 1.6140x; 1.6140x over previous
import functools

import jax
import jax.numpy as jnp
from jax import lax
from jax.experimental import pallas as pl
from jax.experimental.pallas import tpu as pltpu
from jax.experimental.pallas import tpu_sc as plsc

EMBED_DIM = 64
NUM_CORES = 2
NUM_SUBCORES = 16
NUM_WORKERS = NUM_CORES * NUM_SUBCORES

CHUNK = 800


@functools.lru_cache(maxsize=None)
def _make_gather(batch: int, hist: int):
    total_rows = batch * hist
    assert total_rows % (NUM_WORKERS * 2 * CHUNK) == 0
    rows_per_worker = total_rows // NUM_WORKERS
    num_chunks = rows_per_worker // CHUNK
    num_pairs = num_chunks // 2
    mesh = plsc.VectorSubcoreMesh(core_axis_name="c", subcore_axis_name="s")

    @functools.partial(
        pl.kernel,
        mesh=mesh,
        out_type=jax.ShapeDtypeStruct((total_rows, EMBED_DIM), jnp.float32),
        compiler_params=pltpu.CompilerParams(use_tc_tiling_on_sc=False),
        scratch_types=[
            pltpu.VMEM((CHUNK,), jnp.int32),
            pltpu.VMEM((CHUNK,), jnp.int32),
            pltpu.VMEM((CHUNK, EMBED_DIM), jnp.float32),
            pltpu.VMEM((CHUNK, EMBED_DIM), jnp.float32),
            pltpu.SemaphoreType.DMA,
            pltpu.SemaphoreType.DMA,
            pltpu.SemaphoreType.DMA,
            pltpu.SemaphoreType.DMA,
            pltpu.SemaphoreType.DMA,
            pltpu.SemaphoreType.DMA,
        ],
    )
    def gather_kernel(idx_hbm, table_hbm, out_hbm,
                      idx0, idx1, rows0, rows1,
                      isem0, isem1, gsem0, gsem1, osem0, osem1):
        wid = lax.axis_index("s") * NUM_CORES + lax.axis_index("c")
        base = wid * rows_per_worker

        pltpu.async_copy(idx_hbm.at[pl.ds(base, CHUNK)], idx0, isem0)
        pltpu.async_copy(idx_hbm.at[pl.ds(base + CHUNK, CHUNK)], idx1, isem1)

        bufs = ((idx0, rows0, isem0, gsem0, osem0),
                (idx1, rows1, isem1, gsem1, osem1))

        def body(i, _):
            for b, (idx_v, rows_v, isem, gsem, osem) in enumerate(bufs):
                c = 2 * i + b
                off = base + c * CHUNK
                pltpu.make_async_copy(idx_hbm.at[pl.ds(off, CHUNK)], idx_v,
                                      isem).wait()
                @pl.when(i > 0)
                def _():
                    pltpu.make_async_copy(
                        rows_v, out_hbm.at[pl.ds(off - 2 * CHUNK, CHUNK)],
                        osem).wait()
                pltpu.async_copy(table_hbm.at[idx_v], rows_v, gsem).wait()
                @pl.when(c + 2 < num_chunks)
                def _():
                    pltpu.async_copy(
                        idx_hbm.at[pl.ds(off + 2 * CHUNK, CHUNK)], idx_v, isem)
                pltpu.async_copy(rows_v, out_hbm.at[pl.ds(off, CHUNK)], osem)
            return 0

        lax.fori_loop(0, num_pairs, body, 0)

        tail = base + (num_chunks - 2) * CHUNK
        pltpu.make_async_copy(rows0, out_hbm.at[pl.ds(tail, CHUNK)],
                              osem0).wait()
        pltpu.make_async_copy(rows1, out_hbm.at[pl.ds(tail + CHUNK, CHUNK)],
                              osem1).wait()

    return gather_kernel


def kernel(x, table):
    batch, hist = x.shape
    idx = x.reshape(batch * hist).astype(jnp.int32)
    out = _make_gather(batch, hist)(idx, table)
    return out.reshape(batch, hist, EMBED_DIM)
